# baseline (speedup 1.0000x reference)
import functools

import jax
import jax.numpy as jnp
from jax import lax
from jax.experimental import pallas as pl
from jax.experimental.pallas import tpu as pltpu
from jax.experimental.pallas import tpu_sc as plsc

_N_SRC = 4
_SIZE = 262144
_DELAY = 32
_BUF_LEN = _DELAY + 1

_NCH = 64
_CH = _SIZE // _NCH
_NROWS = _N_SRC * _NCH

_info = plsc.get_sparse_core_info()
_NC, _NS = _info.num_cores, _info.num_subcores
_NW = _NC * _NS
_RPW = _NROWS // _NW


def _sc_body(rows_hbm, buf_hbm, out_hbm, idx_v, rows_v, sem):
    wid = lax.axis_index("s") * _NC + lax.axis_index("c")
    base = wid * _RPW
    pltpu.sync_copy(rows_hbm.at[pl.ds(base, _RPW)], idx_v)
    pltpu.async_copy(buf_hbm.at[idx_v], rows_v, sem).wait()
    pltpu.sync_copy(rows_v, out_hbm.at[pl.ds(base, _RPW)])


_sc_gather = functools.partial(
    pl.kernel,
    out_type=jax.ShapeDtypeStruct((_NROWS, _CH), jnp.float32),
    mesh=plsc.VectorSubcoreMesh(core_axis_name="c", subcore_axis_name="s"),
    scratch_types=[
        pltpu.VMEM((_RPW,), jnp.int32),
        pltpu.VMEM((_RPW, _CH), jnp.float32),
        pltpu.SemaphoreType.DMA,
    ],
)(_sc_body)


def kernel(spikes, buffer, ptr):
    del spikes
    read_idx = jnp.asarray((ptr + 1 - _DELAY) % _BUF_LEN, jnp.int32)
    r = jnp.arange(_NROWS, dtype=jnp.int32)
    rows = ((r // _NCH) * _BUF_LEN + read_idx) * _NCH + (r % _NCH)
    buf2d = buffer.reshape(_N_SRC * _BUF_LEN * _NCH, _CH)
    out = _sc_gather(rows, buf2d)
    return out.reshape(_N_SRC, _SIZE)


# speedup vs baseline: 16.1971x; 16.1971x over previous
import functools

import jax
import jax.numpy as jnp
from jax import lax
from jax.experimental import pallas as pl
from jax.experimental.pallas import tpu as pltpu
from jax.experimental.pallas import tpu_sc as plsc

_N_SRC = 4
_SIZE = 262144
_DELAY = 32
_BUF_LEN = _DELAY + 1

_info = plsc.get_sparse_core_info()
_NC, _NS, _NL = _info.num_cores, _info.num_subcores, _info.num_lanes
_NW = _NC * _NS
_PPS = _NW // _N_SRC
_CH = _SIZE // _PPS


def _sc_body(idx_hbm, buf_hbm, out_hbm, idx_v, chunk_v):
    wid = lax.axis_index("s") * _NC + lax.axis_index("c")
    src = wid // _PPS
    off = (wid % _PPS) * _CH
    pltpu.sync_copy(idx_hbm, idx_v)
    slot = idx_v[...][0]
    pltpu.sync_copy(buf_hbm.at[src, slot, pl.ds(off, _CH)], chunk_v)
    pltpu.sync_copy(chunk_v, out_hbm.at[src, pl.ds(off, _CH)])


_sc_slice = functools.partial(
    pl.kernel,
    out_type=jax.ShapeDtypeStruct((_N_SRC, _SIZE), jnp.float32),
    mesh=plsc.VectorSubcoreMesh(core_axis_name="c", subcore_axis_name="s"),
    scratch_types=[
        pltpu.VMEM((_NL,), jnp.int32),
        pltpu.VMEM((_CH,), jnp.float32),
    ],
)(_sc_body)


def kernel(spikes, buffer, ptr):
    del spikes
    read_idx = jnp.asarray((ptr + 1 - _DELAY) % _BUF_LEN, jnp.int32)
    idx_vec = jnp.zeros((_NL,), dtype=jnp.int32).at[0].set(read_idx)
    return _sc_slice(idx_vec, buffer)
